# Initial kernel scaffold; baseline (speedup 1.0000x reference)
import functools

import jax
import jax.numpy as jnp
from jax import lax
from jax.experimental import pallas as pl
from jax.experimental.pallas import tpu as pltpu
from jax.experimental.pallas import tpu_sc as plsc

N = 10000
E = 320000
IN_DIM = 128
HID_DIM = 32

NC = 2
NS = 16
NW = NC * NS
LANES = 16

E_PER_W = E // NW
CHUNK = 80
N_CHUNKS = E_PER_W // CHUNK
ROWS_PER_TILE = N // NS

_mesh = plsc.VectorSubcoreMesh(core_axis_name="c", subcore_axis_name="s")


def _zero_1d(ref, total):
  def body(i, _):
    ref[pl.ds(i * LANES, LANES)] = jnp.zeros((LANES,), ref.dtype)
    return 0
  lax.fori_loop(0, total // LANES, body, 0)


@functools.partial(
    pl.kernel,
    out_type=jax.ShapeDtypeStruct((NW, N), jnp.float32),
    mesh=_mesh,
    scratch_types=[
        pltpu.VMEM((E_PER_W,), jnp.int32),
        pltpu.VMEM((N,), jnp.float32),
    ],
)
def _sc_degree(col_hbm, out_hbm, col_v, acc_v):
  wid = lax.axis_index("s") * NC + lax.axis_index("c")
  pltpu.sync_copy(col_hbm.at[pl.ds(wid * E_PER_W, E_PER_W)], col_v)
  _zero_1d(acc_v, N)
  ones = jnp.ones((LANES,), jnp.float32)

  def body(i, _):
    c = col_v[pl.ds(i * LANES, LANES)]
    plsc.addupdate_scatter(acc_v, [c], ones)
    return 0
  lax.fori_loop(0, E_PER_W // LANES, body, 0)
  pltpu.sync_copy(acc_v, out_hbm.at[wid])


@functools.partial(
    pl.kernel,
    out_type=jax.ShapeDtypeStruct((NC, N, HID_DIM), jnp.float32),
    mesh=_mesh,
    scratch_types=[
        pltpu.VMEM((CHUNK,), jnp.int32),
        pltpu.VMEM((CHUNK,), jnp.int32),
        pltpu.VMEM((CHUNK, HID_DIM), jnp.float32),
        pltpu.VMEM((ROWS_PER_TILE, HID_DIM), jnp.float32),
        pltpu.VMEM_SHARED((N, HID_DIM), jnp.float32),
        pltpu.SemaphoreType.DMA,
    ],
)
def _sc_agg1(row_hbm, col_hbm, y_hbm, out_hbm,
             ridx_v, cidx_v, rows_v, stage_v, agg_sh, sem):
  cid = lax.axis_index("c")
  sid = lax.axis_index("s")
  wid = sid * NC + cid

  def zbody(j, _):
    stage_v[j, pl.ds(0, LANES)] = jnp.zeros((LANES,), jnp.float32)
    stage_v[j, pl.ds(LANES, LANES)] = jnp.zeros((LANES,), jnp.float32)
    return 0
  lax.fori_loop(0, ROWS_PER_TILE, zbody, 0)
  pltpu.sync_copy(stage_v, agg_sh.at[pl.ds(sid * ROWS_PER_TILE, ROWS_PER_TILE)])
  plsc.subcore_barrier()

  def body(k, _):
    base = wid * E_PER_W + k * CHUNK
    pltpu.sync_copy(row_hbm.at[pl.ds(base, CHUNK)], ridx_v)
    pltpu.sync_copy(col_hbm.at[pl.ds(base, CHUNK)], cidx_v)
    pltpu.async_copy(y_hbm.at[ridx_v], rows_v, sem).wait()
    pltpu.sync_copy(rows_v, agg_sh.at[cidx_v], add=True)
    return 0
  lax.fori_loop(0, N_CHUNKS, body, 0)
  plsc.subcore_barrier()

  pltpu.sync_copy(agg_sh.at[pl.ds(sid * ROWS_PER_TILE, ROWS_PER_TILE)], stage_v)
  pltpu.sync_copy(stage_v, out_hbm.at[cid, pl.ds(sid * ROWS_PER_TILE, ROWS_PER_TILE)])


@functools.partial(
    pl.kernel,
    out_type=jax.ShapeDtypeStruct((NW, N), jnp.float32),
    mesh=_mesh,
    scratch_types=[
        pltpu.VMEM((E_PER_W,), jnp.int32),
        pltpu.VMEM((E_PER_W,), jnp.int32),
        pltpu.VMEM((N,), jnp.float32),
        pltpu.VMEM((N,), jnp.float32),
    ],
)
def _sc_agg2(row_hbm, col_hbm, y2_hbm, out_hbm, row_v, col_v, y2_v, acc_v):
  wid = lax.axis_index("s") * NC + lax.axis_index("c")
  pltpu.sync_copy(row_hbm.at[pl.ds(wid * E_PER_W, E_PER_W)], row_v)
  pltpu.sync_copy(col_hbm.at[pl.ds(wid * E_PER_W, E_PER_W)], col_v)
  pltpu.sync_copy(y2_hbm, y2_v)
  _zero_1d(acc_v, N)

  def body(i, _):
    r = row_v[pl.ds(i * LANES, LANES)]
    c = col_v[pl.ds(i * LANES, LANES)]
    v = plsc.load_gather(y2_v, [r])
    plsc.addupdate_scatter(acc_v, [c], v)
    return 0
  lax.fori_loop(0, E_PER_W // LANES, body, 0)
  pltpu.sync_copy(acc_v, out_hbm.at[wid])


def _tc_d_body(p_ref, d_ref):
  deg = jnp.sum(p_ref[...], axis=0, keepdims=True) + 1.0
  d_ref[...] = lax.rsqrt(deg)


def _tc_y_body(x_ref, w1_ref, d_ref, y_ref):
  xw = jnp.dot(x_ref[...], w1_ref[...], preferred_element_type=jnp.float32)
  y_ref[...] = d_ref[...] * xw


def _tc_h_body(a0_ref, a1_ref, y_ref, d_ref, b1_ref, w2_ref, y2_ref):
  agg = a0_ref[...] + a1_ref[...] + y_ref[...]
  h = jnp.maximum(d_ref[...] * agg + b1_ref[...], 0.0)
  hw = jnp.dot(h, w2_ref[...], preferred_element_type=jnp.float32)
  y2_ref[...] = d_ref[...] * hw


def _tc_out_body(p2_ref, y2_ref, d_ref, b2_ref, o_ref):
  agg = jnp.sum(p2_ref[...], axis=0, keepdims=True) + y2_ref[...]
  o_ref[...] = d_ref[...] * agg + b2_ref[...]


def kernel(x, edge_index, W1, b1, W2, b2):
  row = edge_index[0]
  col = edge_index[1]

  deg_part = _sc_degree(col)

  d_row = pl.pallas_call(
      _tc_d_body,
      out_shape=jax.ShapeDtypeStruct((1, N), jnp.float32),
  )(deg_part)
  d_col = d_row.reshape(N, 1)

  y = pl.pallas_call(
      _tc_y_body,
      out_shape=jax.ShapeDtypeStruct((N, HID_DIM), jnp.float32),
  )(x, W1, d_col)

  agg1 = _sc_agg1(row, col, y)

  y2 = pl.pallas_call(
      _tc_h_body,
      out_shape=jax.ShapeDtypeStruct((N, 1), jnp.float32),
  )(agg1[0], agg1[1], y, d_col, b1.reshape(1, HID_DIM), W2)

  p2 = _sc_agg2(row, col, y2.reshape(N))

  o_row = pl.pallas_call(
      _tc_out_body,
      out_shape=jax.ShapeDtypeStruct((1, N), jnp.float32),
  )(p2, y2.reshape(1, N), d_row, b2.reshape(1, 1))

  return o_row.reshape(N, 1)


# speedup vs baseline: 31.1249x; 31.1249x over previous
import functools

import jax
import jax.numpy as jnp
from jax import lax
from jax.experimental import pallas as pl
from jax.experimental.pallas import tpu as pltpu
from jax.experimental.pallas import tpu_sc as plsc

N = 10000
E = 320000
IN_DIM = 128
HID_DIM = 32

NC = 2
NS = 16
NW = NC * NS
LANES = 16

E_PER_W = E // NW
CHUNK = 80
N_CHUNKS = E_PER_W // CHUNK
ROWS_PER_TILE = N // NS

_mesh = plsc.VectorSubcoreMesh(core_axis_name="c", subcore_axis_name="s")


def _zero_1d(ref, total):
  def body(i, _):
    ref[pl.ds(i * LANES, LANES)] = jnp.zeros((LANES,), ref.dtype)
    return 0
  lax.fori_loop(0, total // LANES, body, 0)


@functools.partial(
    pl.kernel,
    out_type=jax.ShapeDtypeStruct((NW, N), jnp.float32),
    mesh=_mesh,
    compiler_params=pltpu.CompilerParams(needs_layout_passes=False, use_tc_tiling_on_sc=False),
    scratch_types=[
        pltpu.VMEM((E_PER_W,), jnp.int32),
        pltpu.VMEM((N,), jnp.float32),
    ],
)
def _sc_degree(col_hbm, out_hbm, col_v, acc_v):
  wid = lax.axis_index("s") * NC + lax.axis_index("c")
  pltpu.sync_copy(col_hbm.at[pl.ds(wid * E_PER_W, E_PER_W)], col_v)
  _zero_1d(acc_v, N)
  ones = jnp.ones((LANES,), jnp.float32)

  def body(i, _):
    c = col_v[pl.ds(i * LANES, LANES)]
    plsc.addupdate_scatter(acc_v, [c], ones)
    return 0
  lax.fori_loop(0, E_PER_W // LANES, body, 0)
  pltpu.sync_copy(acc_v, out_hbm.at[wid])


@functools.partial(
    pl.kernel,
    out_type=jax.ShapeDtypeStruct((NC, N, HID_DIM), jnp.float32),
    mesh=_mesh,
    compiler_params=pltpu.CompilerParams(needs_layout_passes=False, use_tc_tiling_on_sc=False),
    scratch_types=[
        pltpu.VMEM((CHUNK,), jnp.int32),
        pltpu.VMEM((CHUNK,), jnp.int32),
        pltpu.VMEM((CHUNK, HID_DIM), jnp.float32),
        pltpu.VMEM((ROWS_PER_TILE, HID_DIM), jnp.float32),
        pltpu.VMEM_SHARED((N, HID_DIM), jnp.float32),
        pltpu.SemaphoreType.DMA,
    ],
)
def _sc_agg1(row_hbm, col_hbm, y_hbm, out_hbm,
             ridx_v, cidx_v, rows_v, stage_v, agg_sh, sem):
  cid = lax.axis_index("c")
  sid = lax.axis_index("s")
  wid = sid * NC + cid

  def zbody(j, _):
    stage_v[j, pl.ds(0, LANES)] = jnp.zeros((LANES,), jnp.float32)
    stage_v[j, pl.ds(LANES, LANES)] = jnp.zeros((LANES,), jnp.float32)
    return 0
  lax.fori_loop(0, ROWS_PER_TILE, zbody, 0)
  pltpu.sync_copy(stage_v, agg_sh.at[pl.ds(sid * ROWS_PER_TILE, ROWS_PER_TILE)])
  plsc.subcore_barrier()

  def body(k, _):
    base = wid * E_PER_W + k * CHUNK
    pltpu.sync_copy(row_hbm.at[pl.ds(base, CHUNK)], ridx_v)
    pltpu.sync_copy(col_hbm.at[pl.ds(base, CHUNK)], cidx_v)
    pltpu.async_copy(y_hbm.at[ridx_v], rows_v, sem).wait()
    pltpu.sync_copy(rows_v, agg_sh.at[cidx_v], add=True)
    return 0
  lax.fori_loop(0, N_CHUNKS, body, 0)
  plsc.subcore_barrier()

  pltpu.sync_copy(agg_sh.at[pl.ds(sid * ROWS_PER_TILE, ROWS_PER_TILE)], stage_v)
  pltpu.sync_copy(stage_v, out_hbm.at[cid, pl.ds(sid * ROWS_PER_TILE, ROWS_PER_TILE)])


@functools.partial(
    pl.kernel,
    out_type=jax.ShapeDtypeStruct((NW, N), jnp.float32),
    mesh=_mesh,
    compiler_params=pltpu.CompilerParams(needs_layout_passes=False, use_tc_tiling_on_sc=False),
    scratch_types=[
        pltpu.VMEM((E_PER_W,), jnp.int32),
        pltpu.VMEM((E_PER_W,), jnp.int32),
        pltpu.VMEM((N,), jnp.float32),
        pltpu.VMEM((N,), jnp.float32),
    ],
)
def _sc_agg2(row_hbm, col_hbm, y2_hbm, out_hbm, row_v, col_v, y2_v, acc_v):
  wid = lax.axis_index("s") * NC + lax.axis_index("c")
  pltpu.sync_copy(row_hbm.at[pl.ds(wid * E_PER_W, E_PER_W)], row_v)
  pltpu.sync_copy(col_hbm.at[pl.ds(wid * E_PER_W, E_PER_W)], col_v)
  pltpu.sync_copy(y2_hbm, y2_v)
  _zero_1d(acc_v, N)

  def body(i, _):
    r = row_v[pl.ds(i * LANES, LANES)]
    c = col_v[pl.ds(i * LANES, LANES)]
    v = plsc.load_gather(y2_v, [r])
    plsc.addupdate_scatter(acc_v, [c], v)
    return 0
  lax.fori_loop(0, E_PER_W // LANES, body, 0)
  pltpu.sync_copy(acc_v, out_hbm.at[wid])


def _tc_d_body(p_ref, d_ref):
  deg = jnp.sum(p_ref[...], axis=0, keepdims=True) + 1.0
  d_ref[...] = lax.rsqrt(deg)


def _tc_y_body(x_ref, w1_ref, d_ref, y_ref):
  xw = jnp.dot(x_ref[...], w1_ref[...], preferred_element_type=jnp.float32)
  y_ref[...] = d_ref[...] * xw


def _tc_h_body(a0_ref, a1_ref, y_ref, d_ref, b1_ref, w2_ref, y2_ref):
  agg = a0_ref[...] + a1_ref[...] + y_ref[...]
  h = jnp.maximum(d_ref[...] * agg + b1_ref[...], 0.0)
  hw = jnp.dot(h, w2_ref[...], preferred_element_type=jnp.float32)
  y2_ref[...] = d_ref[...] * hw


def _tc_out_body(p2_ref, y2_ref, d_ref, b2_ref, o_ref):
  agg = jnp.sum(p2_ref[...], axis=0, keepdims=True) + y2_ref[...]
  o_ref[...] = d_ref[...] * agg + b2_ref[...]


def kernel(x, edge_index, W1, b1, W2, b2):
  row = edge_index[0]
  col = edge_index[1]

  deg_part = _sc_degree(col)

  d_row = pl.pallas_call(
      _tc_d_body,
      out_shape=jax.ShapeDtypeStruct((1, N), jnp.float32),
  )(deg_part)
  d_col = d_row.reshape(N, 1)

  y = pl.pallas_call(
      _tc_y_body,
      out_shape=jax.ShapeDtypeStruct((N, HID_DIM), jnp.float32),
  )(x, W1, d_col)

  agg1 = _sc_agg1(row, col, y)

  y2 = pl.pallas_call(
      _tc_h_body,
      out_shape=jax.ShapeDtypeStruct((N, 1), jnp.float32),
  )(agg1[0], agg1[1], y, d_col, b1.reshape(1, HID_DIM), W2)

  p2 = _sc_agg2(row, col, y2.reshape(N))

  o_row = pl.pallas_call(
      _tc_out_body,
      out_shape=jax.ShapeDtypeStruct((1, N), jnp.float32),
  )(p2, y2.reshape(1, N), d_row, b2.reshape(1, 1))

  return o_row.reshape(N, 1)


# speedup vs baseline: 67.9185x; 2.1821x over previous
import functools

import jax
import jax.numpy as jnp
from jax import lax
from jax.experimental import pallas as pl
from jax.experimental.pallas import tpu as pltpu
from jax.experimental.pallas import tpu_sc as plsc

N = 10000
E = 320000
IN_DIM = 128
HID_DIM = 32

NC = 2
NS = 16
NW = NC * NS
LANES = 16

E_PER_W = E // NW
CHUNK = 80
N_CHUNKS = E_PER_W // CHUNK
NBUF = 4
ROWS_PER_TILE = N // NS

_mesh = plsc.VectorSubcoreMesh(core_axis_name="c", subcore_axis_name="s")


def _zero_1d(ref, total):
  def body(i, _):
    ref[pl.ds(i * LANES, LANES)] = jnp.zeros((LANES,), ref.dtype)
    return 0
  lax.fori_loop(0, total // LANES, body, 0)


@functools.partial(
    pl.kernel,
    out_type=jax.ShapeDtypeStruct((NW, N), jnp.float32),
    mesh=_mesh,
    compiler_params=pltpu.CompilerParams(needs_layout_passes=False, use_tc_tiling_on_sc=False),
    scratch_types=[
        pltpu.VMEM((E_PER_W,), jnp.int32),
        pltpu.VMEM((N,), jnp.float32),
    ],
)
def _sc_degree(col_hbm, out_hbm, col_v, acc_v):
  wid = lax.axis_index("s") * NC + lax.axis_index("c")
  pltpu.sync_copy(col_hbm.at[pl.ds(wid * E_PER_W, E_PER_W)], col_v)
  _zero_1d(acc_v, N)
  ones = jnp.ones((LANES,), jnp.float32)

  def body(i, _):
    c = col_v[pl.ds(i * LANES, LANES)]
    plsc.addupdate_scatter(acc_v, [c], ones)
    return 0
  lax.fori_loop(0, E_PER_W // LANES, body, 0)
  pltpu.sync_copy(acc_v, out_hbm.at[wid])


@functools.partial(
    pl.kernel,
    out_type=jax.ShapeDtypeStruct((NC, N, HID_DIM), jnp.float32),
    mesh=_mesh,
    compiler_params=pltpu.CompilerParams(needs_layout_passes=False, use_tc_tiling_on_sc=False),
    scratch_types=[
        pltpu.VMEM((N_CHUNKS, CHUNK), jnp.int32),
        pltpu.VMEM((N_CHUNKS, CHUNK), jnp.int32),
        pltpu.VMEM((NBUF, CHUNK, HID_DIM), jnp.float32),
        pltpu.VMEM((ROWS_PER_TILE, HID_DIM), jnp.float32),
        pltpu.VMEM_SHARED((N, HID_DIM), jnp.float32),
        pltpu.SemaphoreType.DMA,
    ],
)
def _sc_agg1(row_hbm, col_hbm, y_hbm, out_hbm,
             ridx_v, cidx_v, rows_v, stage_v, agg_sh, sem):
  cid = lax.axis_index("c")
  sid = lax.axis_index("s")
  wid = sid * NC + cid

  def zbody(j, _):
    stage_v[j, pl.ds(0, LANES)] = jnp.zeros((LANES,), jnp.float32)
    stage_v[j, pl.ds(LANES, LANES)] = jnp.zeros((LANES,), jnp.float32)
    return 0
  lax.fori_loop(0, ROWS_PER_TILE, zbody, 0)
  pltpu.sync_copy(stage_v, agg_sh.at[pl.ds(sid * ROWS_PER_TILE, ROWS_PER_TILE)])

  pltpu.sync_copy(row_hbm.at[wid], ridx_v)
  pltpu.sync_copy(col_hbm.at[wid], cidx_v)
  plsc.subcore_barrier()

  for b in range(NBUF):
    pltpu.async_copy(y_hbm.at[ridx_v.at[b]], rows_v.at[b], sem)

  def body(k, _):
    b = lax.rem(k, NBUF)
    pltpu.make_async_copy(y_hbm.at[ridx_v.at[k]], rows_v.at[b], sem).wait()
    pltpu.sync_copy(rows_v.at[b], agg_sh.at[cidx_v.at[k]], add=True)
    nk = k + NBUF

    @pl.when(nk < N_CHUNKS)
    def _():
      pltpu.async_copy(y_hbm.at[ridx_v.at[nk]], rows_v.at[b], sem)
    return 0
  lax.fori_loop(0, N_CHUNKS, body, 0)
  plsc.subcore_barrier()

  pltpu.sync_copy(agg_sh.at[pl.ds(sid * ROWS_PER_TILE, ROWS_PER_TILE)], stage_v)
  pltpu.sync_copy(stage_v, out_hbm.at[cid, pl.ds(sid * ROWS_PER_TILE, ROWS_PER_TILE)])


@functools.partial(
    pl.kernel,
    out_type=jax.ShapeDtypeStruct((NW, N), jnp.float32),
    mesh=_mesh,
    compiler_params=pltpu.CompilerParams(needs_layout_passes=False, use_tc_tiling_on_sc=False),
    scratch_types=[
        pltpu.VMEM((E_PER_W,), jnp.int32),
        pltpu.VMEM((E_PER_W,), jnp.int32),
        pltpu.VMEM((N,), jnp.float32),
        pltpu.VMEM((N,), jnp.float32),
    ],
)
def _sc_agg2(row_hbm, col_hbm, y2_hbm, out_hbm, row_v, col_v, y2_v, acc_v):
  wid = lax.axis_index("s") * NC + lax.axis_index("c")
  pltpu.sync_copy(row_hbm.at[pl.ds(wid * E_PER_W, E_PER_W)], row_v)
  pltpu.sync_copy(col_hbm.at[pl.ds(wid * E_PER_W, E_PER_W)], col_v)
  pltpu.sync_copy(y2_hbm, y2_v)
  _zero_1d(acc_v, N)

  def body(i, _):
    r = row_v[pl.ds(i * LANES, LANES)]
    c = col_v[pl.ds(i * LANES, LANES)]
    v = plsc.load_gather(y2_v, [r])
    plsc.addupdate_scatter(acc_v, [c], v)
    return 0
  lax.fori_loop(0, E_PER_W // LANES, body, 0)
  pltpu.sync_copy(acc_v, out_hbm.at[wid])


def _tc_d_body(p_ref, d_ref):
  deg = jnp.sum(p_ref[...], axis=0, keepdims=True) + 1.0
  d_ref[...] = lax.rsqrt(deg)


def _tc_y_body(x_ref, w1_ref, d_ref, y_ref):
  xw = jnp.dot(x_ref[...], w1_ref[...], preferred_element_type=jnp.float32)
  y_ref[...] = d_ref[...] * xw


def _tc_h_body(a0_ref, a1_ref, y_ref, d_ref, b1_ref, w2_ref, y2_ref):
  agg = a0_ref[...] + a1_ref[...] + y_ref[...]
  h = jnp.maximum(d_ref[...] * agg + b1_ref[...], 0.0)
  hw = jnp.dot(h, w2_ref[...], preferred_element_type=jnp.float32)
  y2_ref[...] = d_ref[...] * hw


def _tc_out_body(p2_ref, y2_ref, d_ref, b2_ref, o_ref):
  agg = jnp.sum(p2_ref[...], axis=0, keepdims=True) + y2_ref[...]
  o_ref[...] = d_ref[...] * agg + b2_ref[...]


def kernel(x, edge_index, W1, b1, W2, b2):
  row = edge_index[0]
  col = edge_index[1]

  deg_part = _sc_degree(col)

  d_row = pl.pallas_call(
      _tc_d_body,
      out_shape=jax.ShapeDtypeStruct((1, N), jnp.float32),
  )(deg_part)
  d_col = d_row.reshape(N, 1)

  y = pl.pallas_call(
      _tc_y_body,
      out_shape=jax.ShapeDtypeStruct((N, HID_DIM), jnp.float32),
  )(x, W1, d_col)

  row3 = row.reshape(NW, N_CHUNKS, CHUNK)
  col3 = col.reshape(NW, N_CHUNKS, CHUNK)
  agg1 = _sc_agg1(row3, col3, y)

  y2 = pl.pallas_call(
      _tc_h_body,
      out_shape=jax.ShapeDtypeStruct((N, 1), jnp.float32),
  )(agg1[0], agg1[1], y, d_col, b1.reshape(1, HID_DIM), W2)

  p2 = _sc_agg2(row, col, y2.reshape(N))

  o_row = pl.pallas_call(
      _tc_out_body,
      out_shape=jax.ShapeDtypeStruct((1, N), jnp.float32),
  )(p2, y2.reshape(1, N), d_row, b2.reshape(1, 1))

  return o_row.reshape(N, 1)
